# Initial kernel scaffold; baseline (speedup 1.0000x reference)
import jax
import jax.numpy as jnp
from jax.experimental import pallas as pl


def kernel(idx, weight):
    raise NotImplementedError("write your pallas kernel here")


# speedup vs baseline: 1.0783x; 1.0783x over previous
import functools

import jax
import jax.numpy as jnp
from jax import lax
from jax.experimental import pallas as pl
from jax.experimental.pallas import tpu as pltpu
from jax.experimental.pallas import tpu_sc as plsc

D = 32
NC, NS = 2, 16
NW = NC * NS
B = 16384 * 50
CHUNK = 128
BPW = B // NW
NCHUNK = BPW // CHUNK

_mesh = plsc.VectorSubcoreMesh(core_axis_name="c", subcore_axis_name="s")


@functools.partial(
    pl.kernel,
    out_type=jax.ShapeDtypeStruct((B, D), jnp.float32),
    mesh=_mesh,
    scratch_types=[
        pltpu.VMEM((NCHUNK, CHUNK), jnp.int32),
        pltpu.VMEM((CHUNK, D), jnp.float32),
        pltpu.VMEM((CHUNK, D), jnp.float32),
        pltpu.SemaphoreType.DMA,
        pltpu.SemaphoreType.DMA,
        pltpu.SemaphoreType.DMA,
        pltpu.SemaphoreType.DMA,
    ],
    compiler_params=pltpu.CompilerParams(use_tc_tiling_on_sc=False),
)
def _gather_kernel(idx_hbm, w_hbm, out_hbm,
                   idx_v, buf0, buf1, g0, g1, o0, o1):
    wid = lax.axis_index("s") * NC + lax.axis_index("c")
    rbase = wid * BPW
    bufs = (buf0, buf1)
    gsems = (g0, g1)
    osems = (o0, o1)

    pltpu.sync_copy(idx_hbm.at[pl.ds(wid * NCHUNK, NCHUNK)], idx_v)

    pltpu.async_copy(w_hbm.at[idx_v.at[0]], buf0, g0)

    @pl.loop(0, NCHUNK, step=2)
    def _(g):
        for b in range(2):
            c = g + b
            nb = 1 - b
            @pl.when((c >= 1) & (c + 1 < NCHUNK))
            def _():
                pltpu.make_async_copy(
                    bufs[nb], out_hbm.at[pl.ds(rbase + (c - 1) * CHUNK, CHUNK)],
                    osems[nb]).wait()
            @pl.when(c + 1 < NCHUNK)
            def _():
                pltpu.async_copy(w_hbm.at[idx_v.at[c + 1]], bufs[nb], gsems[nb])
            pltpu.make_async_copy(w_hbm.at[idx_v.at[c]], bufs[b], gsems[b]).wait()
            pltpu.async_copy(
                bufs[b], out_hbm.at[pl.ds(rbase + c * CHUNK, CHUNK)], osems[b])

    for b in range(2):
        c = NCHUNK - 2 + b
        pltpu.make_async_copy(
            bufs[b], out_hbm.at[pl.ds(rbase + c * CHUNK, CHUNK)], osems[b]).wait()


def kernel(idx, weight):
    nb, ns = idx.shape
    flat = idx.reshape(nb * ns).astype(jnp.int32)
    idx2d = flat.reshape(NW * NCHUNK, CHUNK)
    out = _gather_kernel(idx2d, weight)
    return out.reshape(nb, ns, D)


# speedup vs baseline: 1.1126x; 1.0319x over previous
import functools

import jax
import jax.numpy as jnp
from jax import lax
from jax.experimental import pallas as pl
from jax.experimental.pallas import tpu as pltpu
from jax.experimental.pallas import tpu_sc as plsc

D = 32
NC, NS = 2, 16
NW = NC * NS
B = 16384 * 50
CHUNK = 128
BPW = B // NW
NCHUNK = BPW // CHUNK
NBUF = 8
LOOK = NBUF - 1

_mesh = plsc.VectorSubcoreMesh(core_axis_name="c", subcore_axis_name="s")


@functools.partial(
    pl.kernel,
    out_type=jax.ShapeDtypeStruct((B, D), jnp.float32),
    mesh=_mesh,
    scratch_types=(
        [pltpu.VMEM((NCHUNK, CHUNK), jnp.int32)]
        + [pltpu.VMEM((CHUNK, D), jnp.float32)] * NBUF
        + [pltpu.SemaphoreType.DMA] * NBUF
        + [pltpu.SemaphoreType.DMA] * NBUF
    ),
    compiler_params=pltpu.CompilerParams(use_tc_tiling_on_sc=False),
)
def _gather_kernel(idx_hbm, w_hbm, out_hbm, idx_v, *rest):
    bufs = rest[:NBUF]
    gsems = rest[NBUF:2 * NBUF]
    osems = rest[2 * NBUF:]
    wid = lax.axis_index("s") * NC + lax.axis_index("c")
    rbase = wid * BPW

    pltpu.sync_copy(idx_hbm.at[pl.ds(wid * NCHUNK, NCHUNK)], idx_v)

    for c in range(LOOK):
        pltpu.async_copy(w_hbm.at[idx_v.at[c]], bufs[c], gsems[c])

    @pl.loop(0, NCHUNK, step=NBUF)
    def _(g):
        for b in range(NBUF):
            c = g + b
            ahead = (b + LOOK) % NBUF
            @pl.when((c >= 1) & (c + LOOK < NCHUNK))
            def _():
                pltpu.make_async_copy(
                    bufs[ahead],
                    out_hbm.at[pl.ds(rbase + (c - 1) * CHUNK, CHUNK)],
                    osems[ahead]).wait()
            @pl.when(c + LOOK < NCHUNK)
            def _():
                pltpu.async_copy(
                    w_hbm.at[idx_v.at[c + LOOK]], bufs[ahead], gsems[ahead])
            pltpu.make_async_copy(
                w_hbm.at[idx_v.at[c]], bufs[b], gsems[b]).wait()
            pltpu.async_copy(
                bufs[b], out_hbm.at[pl.ds(rbase + c * CHUNK, CHUNK)], osems[b])

    for b in range(NBUF):
        c = NCHUNK - NBUF + b
        pltpu.make_async_copy(
            bufs[b], out_hbm.at[pl.ds(rbase + c * CHUNK, CHUNK)], osems[b]).wait()


def kernel(idx, weight):
    nb, ns = idx.shape
    flat = idx.reshape(nb * ns).astype(jnp.int32)
    idx2d = flat.reshape(NW * NCHUNK, CHUNK)
    out = _gather_kernel(idx2d, weight)
    return out.reshape(nb, ns, D)


# speedup vs baseline: 1.5077x; 1.3551x over previous
import functools

import jax
import jax.numpy as jnp
from jax import lax
from jax.experimental import pallas as pl
from jax.experimental.pallas import tpu as pltpu
from jax.experimental.pallas import tpu_sc as plsc

D = 32
NC, NS = 2, 16
NW = NC * NS
NB, NSEQ = 16384, 50
CHUNK = 128
NCH = NSEQ * (NB // CHUNK)
CPW = NCH // NW
BTILES = NB // CHUNK

_mesh = plsc.VectorSubcoreMesh(core_axis_name="c", subcore_axis_name="s")


@functools.partial(
    pl.kernel,
    out_type=jax.ShapeDtypeStruct((NSEQ * 4 * BTILES, 8, CHUNK), jnp.float32),
    mesh=_mesh,
    scratch_types=(
        [pltpu.VMEM((CPW, CHUNK), jnp.int32)]
        + [pltpu.VMEM((CHUNK, D), jnp.float32)] * 2
        + [pltpu.VMEM((4, 8, CHUNK), jnp.float32)] * 2
        + [pltpu.SemaphoreType.DMA] * 2
        + [pltpu.SemaphoreType.DMA] * 2
    ),
    compiler_params=pltpu.CompilerParams(
        use_tc_tiling_on_sc=False, needs_layout_passes=False),
)
def _gather_kernel(idx_hbm, w_hbm, out_hbm,
                   idx_v, row0, row1, t0, t1, g0, g1, o0, o1):
    rows = (row0, row1)
    tbufs = (t0, t1)
    gsems = (g0, g1)
    osems = (o0, o1)
    wid = lax.axis_index("s") * NC + lax.axis_index("c")
    cbase = wid * CPW
    lane = jnp.arange(16, dtype=jnp.int32)

    pltpu.sync_copy(idx_hbm.at[pl.ds(cbase, CPW)], idx_v)

    for k in range(2):
        pltpu.async_copy(w_hbm.at[idx_v.at[k]], rows[k], gsems[k])

    @pl.loop(0, CPW, step=2)
    def _(g):
        for k in range(2):
            c = g + k
            pltpu.make_async_copy(
                w_hbm.at[idx_v.at[c]], rows[k], gsems[k]).wait()
            @pl.when(c >= 2)
            def _():
                for ti in range(4):
                    pltpu.make_async_copy(
                        tbufs[k].at[ti], out_hbm.at[ti], osems[k]).wait()
            for d in range(D):
                col = jnp.full((16,), d, jnp.int32)
                for grp in range(8):
                    vec = plsc.load_gather(
                        rows[k], [lane + (16 * grp), col])
                    tbufs[k][d // 8, d % 8, pl.ds(16 * grp, 16)] = vec
            gg = cbase + c
            s = gg // BTILES
            tj = gg - s * BTILES
            rb = s * (4 * BTILES) + tj
            for ti in range(4):
                pltpu.async_copy(
                    tbufs[k].at[ti], out_hbm.at[rb + ti * BTILES], osems[k])
            @pl.when(c + 2 < CPW)
            def _():
                pltpu.async_copy(
                    w_hbm.at[idx_v.at[c + 2]], rows[k], gsems[k])

    for k in range(2):
        for ti in range(4):
            pltpu.make_async_copy(
                tbufs[k].at[ti], out_hbm.at[ti], osems[k]).wait()


def kernel(idx, weight):
    idx2d = idx.T.astype(jnp.int32).reshape(NCH, CHUNK)
    out = _gather_kernel(idx2d, weight)
    out5 = out.reshape(NSEQ, 4, BTILES, 8, CHUNK)
    return out5.transpose(2, 4, 0, 1, 3).reshape(NB, NSEQ, D)


# speedup vs baseline: 1.5124x; 1.0031x over previous
import functools

import jax
import jax.numpy as jnp
from jax import lax
from jax.experimental import pallas as pl
from jax.experimental.pallas import tpu as pltpu
from jax.experimental.pallas import tpu_sc as plsc

D = 32
NC, NS = 2, 16
NW = NC * NS
NB, NSEQ = 16384, 50
CHUNK = 128
NCH = NSEQ * (NB // CHUNK)
CPW = NCH // NW
BTILES = NB // CHUNK

_mesh = plsc.VectorSubcoreMesh(core_axis_name="c", subcore_axis_name="s")


@functools.partial(
    pl.kernel,
    out_type=jax.ShapeDtypeStruct((NSEQ * 4 * BTILES, 8, CHUNK), jnp.float32),
    mesh=_mesh,
    scratch_types=(
        [pltpu.VMEM((CPW, CHUNK), jnp.int32)]
        + [pltpu.VMEM((CHUNK, D), jnp.float32)] * 2
        + [pltpu.VMEM((4, 8, CHUNK), jnp.float32)] * 2
        + [pltpu.SemaphoreType.DMA] * 2
        + [pltpu.SemaphoreType.DMA] * 2
    ),
    compiler_params=pltpu.CompilerParams(
        use_tc_tiling_on_sc=False, needs_layout_passes=False,
        disable_bounds_checks=True),
)
def _gather_kernel(idx_hbm, w_hbm, out_hbm,
                   idx_v, row0, row1, t0, t1, g0, g1, o0, o1):
    rows = (row0, row1)
    tbufs = (t0, t1)
    gsems = (g0, g1)
    osems = (o0, o1)
    wid = lax.axis_index("s") * NC + lax.axis_index("c")
    cbase = wid * CPW
    base = jnp.arange(16, dtype=jnp.int32)
    cols = [jnp.full((16,), d, jnp.int32) for d in range(D)]

    pltpu.sync_copy(idx_hbm.at[pl.ds(cbase, CPW)], idx_v)

    for k in range(2):
        pltpu.async_copy(w_hbm.at[idx_v.at[k]], rows[k], gsems[k])

    @pl.loop(0, CPW, step=2)
    def _(g):
        for k in range(2):
            c = g + k
            pltpu.make_async_copy(
                w_hbm.at[idx_v.at[c]], rows[k], gsems[k]).wait()
            @pl.when(c >= 2)
            def _():
                for ti in range(4):
                    pltpu.make_async_copy(
                        tbufs[k].at[ti], out_hbm.at[ti], osems[k]).wait()
            for d in range(D):
                for grp in range(8):
                    vec = plsc.load_gather(
                        rows[k], [base + 16 * grp, cols[d]])
                    tbufs[k][d // 8, d % 8, pl.ds(16 * grp, 16)] = vec
            gg = cbase + c
            s = gg // BTILES
            tj = gg - s * BTILES
            rb = s * (4 * BTILES) + tj
            for ti in range(4):
                pltpu.async_copy(
                    tbufs[k].at[ti], out_hbm.at[rb + ti * BTILES], osems[k])
            @pl.when(c + 2 < CPW)
            def _():
                pltpu.async_copy(
                    w_hbm.at[idx_v.at[c + 2]], rows[k], gsems[k])

    for k in range(2):
        for ti in range(4):
            pltpu.make_async_copy(
                tbufs[k].at[ti], out_hbm.at[ti], osems[k]).wait()


def kernel(idx, weight):
    idx2d = idx.T.astype(jnp.int32).reshape(NCH, CHUNK)
    out = _gather_kernel(idx2d, weight)
    out5 = out.reshape(NSEQ, 4, BTILES, 8, CHUNK)
    return out5.transpose(2, 4, 0, 1, 3).reshape(NB, NSEQ, D)


# speedup vs baseline: 2.2141x; 1.4639x over previous
import functools

import jax
import jax.numpy as jnp
from jax import lax
from jax.experimental import pallas as pl
from jax.experimental.pallas import tpu as pltpu
from jax.experimental.pallas import tpu_sc as plsc

D = 32
NC, NS = 2, 16
NW = NC * NS
NB, NSEQ = 16384, 50
CHUNK = 128
NCH = NSEQ * (NB // CHUNK)
CPW = NCH // NW
BTILES = NB // CHUNK

_mesh = plsc.VectorSubcoreMesh(core_axis_name="c", subcore_axis_name="s")


@functools.partial(
    pl.kernel,
    out_type=jax.ShapeDtypeStruct((NSEQ * 4 * BTILES, 8, CHUNK), jnp.float32),
    mesh=_mesh,
    scratch_types=(
        [pltpu.VMEM((CPW, CHUNK), jnp.int32)]
        + [pltpu.VMEM((CHUNK, D), jnp.float32)] * 2
        + [pltpu.VMEM((4, 8, CHUNK), jnp.float32)] * 2
        + [pltpu.SemaphoreType.DMA] * 2
        + [pltpu.SemaphoreType.DMA] * 2
    ),
    compiler_params=pltpu.CompilerParams(
        use_tc_tiling_on_sc=False, needs_layout_passes=False,
        disable_bounds_checks=True),
)
def _gather_kernel(idx_hbm, w_hbm, out_hbm,
                   idx_v, row0, row1, t0, t1, g0, g1, o0, o1):
    rows = (row0, row1)
    tbufs = (t0, t1)
    gsems = (g0, g1)
    osems = (o0, o1)
    wid = lax.axis_index("s") * NC + lax.axis_index("c")
    cbase = wid * CPW
    base = jnp.arange(16, dtype=jnp.int32)
    rowvecs = [base + 16 * g for g in range(8)]

    pltpu.sync_copy(idx_hbm.at[pl.ds(cbase, CPW)], idx_v)

    for k in range(2):
        pltpu.async_copy(w_hbm.at[idx_v.at[k]], rows[k], gsems[k])

    @pl.loop(0, CPW, step=2)
    def _(g):
        for k in range(2):
            c = g + k
            pltpu.make_async_copy(
                w_hbm.at[idx_v.at[c]], rows[k], gsems[k]).wait()
            @pl.when(c >= 2)
            def _():
                for ti in range(4):
                    pltpu.make_async_copy(
                        tbufs[k].at[ti], out_hbm.at[ti], osems[k]).wait()
            rk = rows[k]
            tk = tbufs[k]

            @plsc.parallel_loop(0, D, unroll=4)
            def _(d):
                col = jnp.full((16,), 0, jnp.int32) + d
                ti = d // 8
                dd = d - ti * 8
                for grp in range(8):
                    vec = plsc.load_gather(rk, [rowvecs[grp], col])
                    tk[ti, dd, pl.ds(16 * grp, 16)] = vec
            gg = cbase + c
            s = gg // BTILES
            tj = gg - s * BTILES
            rb = s * (4 * BTILES) + tj
            for ti in range(4):
                pltpu.async_copy(
                    tbufs[k].at[ti], out_hbm.at[rb + ti * BTILES], osems[k])
            @pl.when(c + 2 < CPW)
            def _():
                pltpu.async_copy(
                    w_hbm.at[idx_v.at[c + 2]], rows[k], gsems[k])

    for k in range(2):
        for ti in range(4):
            pltpu.make_async_copy(
                tbufs[k].at[ti], out_hbm.at[ti], osems[k]).wait()


def kernel(idx, weight):
    idx2d = idx.T.astype(jnp.int32).reshape(NCH, CHUNK)
    out = _gather_kernel(idx2d, weight)
    out5 = out.reshape(NSEQ, 4, BTILES, 8, CHUNK)
    return out5.transpose(2, 4, 0, 1, 3).reshape(NB, NSEQ, D)
